# Initial kernel scaffold; baseline (speedup 1.0000x reference)
import jax
import jax.numpy as jnp
from jax.experimental import pallas as pl


def kernel(input_ids, position_ids, word_table, pos_table):
    raise NotImplementedError("write your pallas kernel here")


# speedup vs baseline: 1.5319x; 1.5319x over previous
import functools

import jax
import jax.numpy as jnp
from jax import lax
from jax.experimental import pallas as pl
from jax.experimental.pallas import tpu as pltpu
from jax.experimental.pallas import tpu_sc as plsc

BATCH = 1024
SEQ = 200
EMBED_DIM = 64
N = BATCH * SEQ

NC, NS = 2, 16
NW = NC * NS
ROWS_PER_W = N // NW
IDX_W = 128
IDX_ROWS_PER_W = ROWS_PER_W // IDX_W
IDX_ROWS_PER_G = 5
GROUP_ROWS = IDX_ROWS_PER_G * IDX_W
NGROUPS = ROWS_PER_W // GROUP_ROWS


def _emb_body(wids_hbm, pids_hbm, wtab_hbm, ptab_hbm, out_hbm,
              widx_v, pidx_v, wrows_v, prows_v, sem):
    c = lax.axis_index("c")
    s = lax.axis_index("s")
    wid = s * NC + c
    pltpu.sync_copy(wids_hbm.at[wid], widx_v)
    pltpu.sync_copy(pids_hbm.at[wid], pidx_v)
    row_base = wid * ROWS_PER_W

    @pl.loop(0, NGROUPS)
    def _group(g):
        copies = []
        for j in range(IDX_ROWS_PER_G):
            copies.append(pltpu.async_copy(
                wtab_hbm.at[widx_v.at[g * IDX_ROWS_PER_G + j]],
                wrows_v.at[pl.ds(j * IDX_W, IDX_W)], sem))
            copies.append(pltpu.async_copy(
                ptab_hbm.at[pidx_v.at[g * IDX_ROWS_PER_G + j]],
                prows_v.at[pl.ds(j * IDX_W, IDX_W)], sem))
        for cp in copies:
            cp.wait()
        dst = pl.ds(row_base + g * GROUP_ROWS, GROUP_ROWS)
        pltpu.sync_copy(wrows_v, out_hbm.at[dst, 0])
        pltpu.sync_copy(prows_v, out_hbm.at[dst, 1])


_emb_call = functools.partial(
    pl.kernel,
    out_type=jax.ShapeDtypeStruct((N, 2, EMBED_DIM), jnp.float32),
    mesh=plsc.VectorSubcoreMesh(core_axis_name="c", subcore_axis_name="s"),
    compiler_params=pltpu.CompilerParams(use_tc_tiling_on_sc=False),
    scratch_types=[
        pltpu.VMEM((IDX_ROWS_PER_W, IDX_W), jnp.int32),
        pltpu.VMEM((IDX_ROWS_PER_W, IDX_W), jnp.int32),
        pltpu.VMEM((GROUP_ROWS, EMBED_DIM), jnp.float32),
        pltpu.VMEM((GROUP_ROWS, EMBED_DIM), jnp.float32),
        pltpu.SemaphoreType.DMA,
    ],
)(_emb_body)


def kernel(input_ids, position_ids, word_table, pos_table):
    wids = input_ids.astype(jnp.int32).reshape(NW, IDX_ROWS_PER_W, IDX_W)
    pids = position_ids.astype(jnp.int32).reshape(NW, IDX_ROWS_PER_W, IDX_W)
    out = _emb_call(wids, pids, word_table, pos_table)
    return out.reshape(BATCH, SEQ, 2 * EMBED_DIM)


# speedup vs baseline: 1.5357x; 1.0025x over previous
import functools

import jax
import jax.numpy as jnp
from jax import lax
from jax.experimental import pallas as pl
from jax.experimental.pallas import tpu as pltpu
from jax.experimental.pallas import tpu_sc as plsc

BATCH = 1024
SEQ = 200
EMBED_DIM = 64
N = BATCH * SEQ

NC, NS = 2, 16
NW = NC * NS
ROWS_PER_W = N // NW
IDX_W = 128
IDX_ROWS_PER_W = ROWS_PER_W // IDX_W
IDX_ROWS_PER_G = 5
GROUP_ROWS = IDX_ROWS_PER_G * IDX_W
NGROUPS = ROWS_PER_W // GROUP_ROWS


def _emb_body(wids_hbm, pids_hbm, wtab_hbm, ptab_hbm, out_hbm,
              widx_v, pidx_v, wrows_v, prows_v, sem):
    c = lax.axis_index("c")
    s = lax.axis_index("s")
    wid = s * NC + c
    pltpu.sync_copy(wids_hbm.at[wid], widx_v)
    pltpu.sync_copy(pids_hbm.at[wid], pidx_v)
    row_base = wid * ROWS_PER_W

    @pl.loop(0, NGROUPS)
    def _group(g):
        copies = []
        for j in range(IDX_ROWS_PER_G):
            copies.append(pltpu.async_copy(
                wtab_hbm.at[widx_v.at[g * IDX_ROWS_PER_G + j]],
                wrows_v.at[pl.ds(j * IDX_W, IDX_W)], sem))
            copies.append(pltpu.async_copy(
                ptab_hbm.at[pidx_v.at[g * IDX_ROWS_PER_G + j]],
                prows_v.at[pl.ds(j * IDX_W, IDX_W)], sem))
        for cp in copies:
            cp.wait()
        dst = pl.ds(row_base + g * GROUP_ROWS, GROUP_ROWS)
        pltpu.sync_copy(wrows_v, out_hbm.at[dst, pl.ds(0, EMBED_DIM)])
        pltpu.sync_copy(prows_v, out_hbm.at[dst, pl.ds(EMBED_DIM, EMBED_DIM)])


_emb_call = functools.partial(
    pl.kernel,
    out_type=jax.ShapeDtypeStruct((N, 2 * EMBED_DIM), jnp.float32),
    mesh=plsc.VectorSubcoreMesh(core_axis_name="c", subcore_axis_name="s"),
    compiler_params=pltpu.CompilerParams(use_tc_tiling_on_sc=False),
    scratch_types=[
        pltpu.VMEM((IDX_ROWS_PER_W, IDX_W), jnp.int32),
        pltpu.VMEM((IDX_ROWS_PER_W, IDX_W), jnp.int32),
        pltpu.VMEM((GROUP_ROWS, EMBED_DIM), jnp.float32),
        pltpu.VMEM((GROUP_ROWS, EMBED_DIM), jnp.float32),
        pltpu.SemaphoreType.DMA,
    ],
)(_emb_body)


def kernel(input_ids, position_ids, word_table, pos_table):
    wids = input_ids.astype(jnp.int32).reshape(NW, IDX_ROWS_PER_W, IDX_W)
    pids = position_ids.astype(jnp.int32).reshape(NW, IDX_ROWS_PER_W, IDX_W)
    out = _emb_call(wids, pids, word_table, pos_table)
    return out.reshape(BATCH, SEQ, 2 * EMBED_DIM)
